# Initial kernel scaffold; baseline (speedup 1.0000x reference)
import jax
import jax.numpy as jnp
from jax.experimental import pallas as pl


def kernel(x, edge_values, label, W1, W2, u_param, w_omega, b_omega, u_omega, edge_index, mask):
    raise NotImplementedError("write your pallas kernel here")


# speedup vs baseline: 3.4368x; 3.4368x over previous
import functools

import jax
import jax.numpy as jnp
from jax import lax
from jax.experimental import pallas as pl
from jax.experimental.pallas import tpu as pltpu
from jax.experimental.pallas import tpu_sc as plsc

_NC = 2
_NS = 16
_NW = _NC * _NS


def _mm_body(x_ref, w_ref, o_ref):
    o_ref[...] = jnp.dot(x_ref[...], w_ref[...],
                         preferred_element_type=jnp.float32)


def _matmul(x, w, block_rows):
    n, k = x.shape
    m = w.shape[1]
    return pl.pallas_call(
        _mm_body,
        grid=(n // block_rows,),
        in_specs=[pl.BlockSpec((block_rows, k), lambda i: (i, 0)),
                  pl.BlockSpec((k, m), lambda i: (0, 0))],
        out_specs=pl.BlockSpec((block_rows, m), lambda i: (i, 0)),
        out_shape=jax.ShapeDtypeStruct((n, m), jnp.float32),
    )(x, w)


def _norm_mm_body(p_ref, w_ref, o_ref):
    s = p_ref[0] + p_ref[1]
    mean = jnp.mean(s, axis=0, keepdims=True)
    var = jnp.mean(jnp.square(s - mean), axis=0, keepdims=True)
    h = jnp.maximum((s - mean) / jnp.sqrt(var + 0.001), 0.0)
    o_ref[...] = jnp.dot(h, w_ref[...], preferred_element_type=jnp.float32)


def _tail_body(md_ref, ml_ref, u_ref, loss_ref, acc_ref):
    md = md_ref[...]
    ml = ml_ref[...][:, :2]
    z = jnp.dot(md, u_ref[...], preferred_element_type=jnp.float32)
    zm = jnp.max(z, axis=1, keepdims=True)
    ez = jnp.exp(z - zm)
    sm = ez / jnp.sum(ez, axis=1, keepdims=True)
    loss_ref[0, 0] = -jnp.sum(jnp.log(jax.nn.sigmoid(ml * sm)))
    pred = sm[:, 1] > sm[:, 0]
    tru = ml[:, 1] > ml[:, 0]
    acc_ref[0, 0] = jnp.mean((pred == tru).astype(jnp.float32))


@functools.lru_cache(maxsize=None)
def _make_segsum(n_nodes, n_edges, d):
    epw = n_edges // _NW
    c_sz = 80
    nch = epw // c_sz
    rpt = 640
    rlast = n_nodes - 15 * rpt
    mesh = plsc.VectorSubcoreMesh(core_axis_name="c", subcore_axis_name="s")

    @functools.partial(
        pl.kernel,
        out_type=jax.ShapeDtypeStruct((_NC, n_nodes, d), jnp.float32),
        mesh=mesh,
        compiler_params=pltpu.CompilerParams(use_tc_tiling_on_sc=False),
        scratch_types=[
            pltpu.VMEM((c_sz,), jnp.int32),
            pltpu.VMEM((1, c_sz), jnp.int32),
            pltpu.VMEM((c_sz,), jnp.float32),
            pltpu.VMEM((c_sz, d), jnp.float32),
            pltpu.VMEM_SHARED((n_nodes, d), jnp.float32),
            pltpu.SemaphoreType.DMA,
        ],
    )
    def seg(xw, src, dst, ev, out, srcb, dstb, evb, rows, acc, sem):
        c = lax.axis_index("c")
        s = lax.axis_index("s")
        wid = s * _NC + c

        def zfill(i, carry):
            def zlane(j, carry2):
                rows[i, pl.ds(j * 16, 16)] = jnp.zeros((16,), jnp.float32)
                return carry2
            return lax.fori_loop(0, d // 16, zlane, carry)
        lax.fori_loop(0, c_sz, zfill, 0)

        nz = jnp.where(s < 15, rpt // c_sz, rlast // c_sz)

        def zcopy(i, carry):
            pltpu.sync_copy(rows, acc.at[pl.ds(s * rpt + i * c_sz, c_sz)])
            return carry
        lax.fori_loop(0, nz, zcopy, 0)
        plsc.subcore_barrier()

        base = wid * epw

        def chunk(k, carry):
            off = base + k * c_sz
            pltpu.sync_copy(src.at[pl.ds(off, c_sz)], srcb)
            pltpu.sync_copy(dst.at[pl.ds(off, c_sz)], dstb.at[0])
            pltpu.sync_copy(ev.at[pl.ds(off, c_sz)], evb)
            pltpu.async_copy(xw.at[srcb], rows, sem).wait()

            def escale(g, carry2):
                evv = evb[pl.ds(g * 16, 16)]
                for lane in range(16):
                    e = g * 16 + lane
                    bv = jnp.full((16,), evv[lane])
                    for j in range(d // 16):
                        rows[e, pl.ds(j * 16, 16)] = (
                            rows[e, pl.ds(j * 16, 16)] * bv)
                return carry2
            lax.fori_loop(0, c_sz // 16, escale, 0)

            pltpu.sync_copy(rows, acc.at[dstb.at[0]], add=True)
            return carry
        lax.fori_loop(0, nch, chunk, 0)
        plsc.subcore_barrier()

        @pl.when(s < 15)
        def _():
            pltpu.sync_copy(acc.at[pl.ds(s * rpt, rpt)],
                            out.at[c, pl.ds(s * rpt, rpt)])

        @pl.when(s == 15)
        def _():
            pltpu.sync_copy(acc.at[pl.ds(15 * rpt, rlast)],
                            out.at[c, pl.ds(15 * rpt, rlast)])

    return seg


@functools.lru_cache(maxsize=None)
def _make_maskgather(n_nodes, n_train, d, dl):
    tiles = 25
    mp = n_train // tiles
    mesh = plsc.VectorSubcoreMesh(core_axis_name="c", subcore_axis_name="s")

    @functools.partial(
        pl.kernel,
        out_type=(jax.ShapeDtypeStruct((n_train, d), jnp.float32),
                  jax.ShapeDtypeStruct((n_train, dl), jnp.float32)),
        mesh=mesh,
        compiler_params=pltpu.CompilerParams(use_tc_tiling_on_sc=False),
        scratch_types=[
            pltpu.VMEM((mp,), jnp.int32),
            pltpu.VMEM((mp, d), jnp.float32),
            pltpu.VMEM((mp, d), jnp.float32),
            pltpu.VMEM((mp, dl), jnp.float32),
            pltpu.SemaphoreType.DMA,
        ],
    )
    def mg(q0, q1, lab, mask, md_out, ml_out, mb, r0, r1, lb, sem):
        c = lax.axis_index("c")
        s = lax.axis_index("s")
        wid = s * _NC + c

        @pl.when(wid < tiles)
        def _():
            off = wid * mp
            pltpu.sync_copy(mask.at[pl.ds(off, mp)], mb)
            pltpu.async_copy(q0.at[mb], r0, sem).wait()
            pltpu.async_copy(q1.at[mb], r1, sem).wait()

            def addb(e, carry):
                for j in range(d // 16):
                    r0[e, pl.ds(j * 16, 16)] = (r0[e, pl.ds(j * 16, 16)]
                                                + r1[e, pl.ds(j * 16, 16)])
                return carry
            lax.fori_loop(0, mp, addb, 0)

            pltpu.sync_copy(r0, md_out.at[pl.ds(off, mp)])
            pltpu.async_copy(lab.at[mb], lb, sem).wait()
            pltpu.sync_copy(lb, ml_out.at[pl.ds(off, mp)])

    return mg


def kernel(x, edge_values, label, W1, W2, u_param, w_omega, b_omega, u_omega,
           edge_index, mask):
    n, d_in = x.shape
    d_out = W2.shape[1]
    n_edges = edge_index.shape[1]
    n_train = mask.shape[0]

    src = edge_index[0]
    dst = edge_index[1]
    labp = jnp.pad(label, ((0, 0), (0, 16 - label.shape[1])))

    xw1 = _matmul(x, W1, 1000)
    p1 = _make_segsum(n, n_edges, W1.shape[1])(xw1, src, dst, edge_values)

    xw2 = pl.pallas_call(
        _norm_mm_body,
        out_shape=jax.ShapeDtypeStruct((n, d_out), jnp.float32),
    )(p1, W2)

    p2 = _make_segsum(n, n_edges, d_out)(xw2, src, dst, edge_values)

    md, ml = _make_maskgather(n, n_train, d_out, 16)(p2[0], p2[1], labp, mask)

    loss, acc = pl.pallas_call(
        _tail_body,
        out_shape=(jax.ShapeDtypeStruct((1, 1), jnp.float32),
                   jax.ShapeDtypeStruct((1, 1), jnp.float32)),
        out_specs=(pl.BlockSpec(memory_space=pltpu.SMEM),
                   pl.BlockSpec(memory_space=pltpu.SMEM)),
    )(md, ml, u_param)

    return loss[0, 0], acc[0, 0]


# speedup vs baseline: 7.4049x; 2.1546x over previous
import functools

import jax
import jax.numpy as jnp
from jax import lax
from jax.experimental import pallas as pl
from jax.experimental.pallas import tpu as pltpu
from jax.experimental.pallas import tpu_sc as plsc

_NC = 2
_NS = 16
_NW = _NC * _NS


def _mm_body(x_ref, w_ref, o_ref):
    o_ref[...] = jnp.dot(x_ref[...], w_ref[...],
                         preferred_element_type=jnp.float32)


def _matmul(x, w, block_rows):
    n, k = x.shape
    m = w.shape[1]
    return pl.pallas_call(
        _mm_body,
        grid=(n // block_rows,),
        in_specs=[pl.BlockSpec((block_rows, k), lambda i: (i, 0)),
                  pl.BlockSpec((k, m), lambda i: (0, 0))],
        out_specs=pl.BlockSpec((block_rows, m), lambda i: (i, 0)),
        out_shape=jax.ShapeDtypeStruct((n, m), jnp.float32),
    )(x, w)


def _norm_mm_body(p_ref, w_ref, o_ref):
    s = p_ref[0] + p_ref[1]
    mean = jnp.mean(s, axis=0, keepdims=True)
    var = jnp.mean(jnp.square(s - mean), axis=0, keepdims=True)
    h = jnp.maximum((s - mean) / jnp.sqrt(var + 0.001), 0.0)
    o_ref[...] = jnp.dot(h, w_ref[...], preferred_element_type=jnp.float32)


def _tail_body(md_ref, ml_ref, u_ref, loss_ref, acc_ref):
    md = md_ref[...]
    ml = ml_ref[...][:, :2]
    z = jnp.dot(md, u_ref[...], preferred_element_type=jnp.float32)
    zm = jnp.max(z, axis=1, keepdims=True)
    ez = jnp.exp(z - zm)
    sm = ez / jnp.sum(ez, axis=1, keepdims=True)
    loss_ref[0, 0] = -jnp.sum(jnp.log(jax.nn.sigmoid(ml * sm)))
    pred = sm[:, 1] > sm[:, 0]
    tru = ml[:, 1] > ml[:, 0]
    acc_ref[0, 0] = jnp.mean((pred == tru).astype(jnp.float32))


@functools.lru_cache(maxsize=None)
def _make_segsum(n_nodes, n_edges, d):
    epw = n_edges // _NW
    c_sz = 80
    nch = epw // c_sz
    rpt = 640
    rlast = n_nodes - 15 * rpt
    mesh = plsc.VectorSubcoreMesh(core_axis_name="c", subcore_axis_name="s")

    @functools.partial(
        pl.kernel,
        out_type=jax.ShapeDtypeStruct((_NC, n_nodes, d), jnp.float32),
        mesh=mesh,
        compiler_params=pltpu.CompilerParams(use_tc_tiling_on_sc=False),
        scratch_types=[
            pltpu.VMEM((epw,), jnp.int32),
            pltpu.VMEM((nch, c_sz), jnp.int32),
            pltpu.VMEM((epw,), jnp.float32),
            pltpu.VMEM((2, c_sz, d), jnp.float32),
            pltpu.VMEM_SHARED((n_nodes, d), jnp.float32),
            pltpu.SemaphoreType.DMA,
            pltpu.SemaphoreType.DMA,
            pltpu.SemaphoreType.DMA,
            pltpu.SemaphoreType.DMA,
            pltpu.SemaphoreType.DMA,
        ],
    )
    def seg(xw, src, dst2, ev, out, srcv, dstv, evv, rows, acc,
            lsem, gsem0, gsem1, ssem0, ssem1):
        c = lax.axis_index("c")
        s = lax.axis_index("s")
        wid = s * _NC + c
        base = wid * epw
        gsems = (gsem0, gsem1)
        ssems = (ssem0, ssem1)

        l0 = pltpu.async_copy(src.at[pl.ds(base, epw)], srcv, lsem)
        l1 = pltpu.async_copy(ev.at[pl.ds(base, epw)], evv, lsem)
        l2 = pltpu.async_copy(dst2.at[pl.ds(wid * nch, nch)], dstv, lsem)

        def zfill(i, carry):
            def zlane(j, carry2):
                rows[0, i, pl.ds(j * 16, 16)] = jnp.zeros((16,), jnp.float32)
                return carry2
            return lax.fori_loop(0, d // 16, zlane, carry)
        lax.fori_loop(0, c_sz, zfill, 0)

        nz = jnp.where(s < 15, rpt // c_sz, rlast // c_sz)

        def zcopy(i, carry):
            pltpu.sync_copy(rows.at[0],
                            acc.at[pl.ds(s * rpt + i * c_sz, c_sz)])
            return carry
        lax.fori_loop(0, nz, zcopy, 0)

        l0.wait()
        l1.wait()
        l2.wait()

        def start_gather(k, b):
            pltpu.async_copy(xw.at[srcv.at[pl.ds(k * c_sz, c_sz)]],
                             rows.at[b], gsems[b])

        def wait_gather(k, b):
            pltpu.make_async_copy(xw.at[srcv.at[pl.ds(k * c_sz, c_sz)]],
                                  rows.at[b], gsems[b]).wait()

        def start_scatter(k, b):
            pltpu.async_copy(rows.at[b], acc.at[dstv.at[k]], ssems[b],
                             add=True)

        def wait_scatter(k, b):
            pltpu.make_async_copy(rows.at[b], acc.at[dstv.at[k]],
                                  ssems[b]).wait()

        start_gather(0, 0)
        plsc.subcore_barrier()

        def do_step(b, k):
            wait_gather(k, b)
            b1 = 1 - b

            @pl.when(k >= 1)
            def _():
                wait_scatter(k - 1, b1)

            @pl.when(k + 1 < nch)
            def _():
                start_gather(k + 1, b1)

            def escale(g, carry2):
                evs = evv[pl.ds(k * c_sz + g * 16, 16)]
                for lane in range(16):
                    e = g * 16 + lane
                    bv = jnp.full((16,), evs[lane])
                    for j in range(d // 16):
                        rows[b, e, pl.ds(j * 16, 16)] = (
                            rows[b, e, pl.ds(j * 16, 16)] * bv)
                return carry2
            lax.fori_loop(0, c_sz // 16, escale, 0)

            start_scatter(k, b)

        def step(k, carry):
            @pl.when(k % 2 == 0)
            def _():
                do_step(0, k)

            @pl.when(k % 2 == 1)
            def _():
                do_step(1, k)
            return carry
        lax.fori_loop(0, nch, step, 0)

        wait_scatter(nch - 1, (nch - 1) % 2)
        plsc.subcore_barrier()

        @pl.when(s < 15)
        def _():
            pltpu.sync_copy(acc.at[pl.ds(s * rpt, rpt)],
                            out.at[c, pl.ds(s * rpt, rpt)])

        @pl.when(s == 15)
        def _():
            pltpu.sync_copy(acc.at[pl.ds(15 * rpt, rlast)],
                            out.at[c, pl.ds(15 * rpt, rlast)])

    return seg


@functools.lru_cache(maxsize=None)
def _make_maskgather(n_nodes, n_train, d, dl):
    tiles = 25
    mp = n_train // tiles
    mesh = plsc.VectorSubcoreMesh(core_axis_name="c", subcore_axis_name="s")

    @functools.partial(
        pl.kernel,
        out_type=(jax.ShapeDtypeStruct((n_train, d), jnp.float32),
                  jax.ShapeDtypeStruct((n_train, dl), jnp.float32)),
        mesh=mesh,
        compiler_params=pltpu.CompilerParams(use_tc_tiling_on_sc=False),
        scratch_types=[
            pltpu.VMEM((mp,), jnp.int32),
            pltpu.VMEM((mp, d), jnp.float32),
            pltpu.VMEM((mp, d), jnp.float32),
            pltpu.VMEM((mp, dl), jnp.float32),
            pltpu.SemaphoreType.DMA,
        ],
    )
    def mg(q0, q1, lab, mask, md_out, ml_out, mb, r0, r1, lb, sem):
        c = lax.axis_index("c")
        s = lax.axis_index("s")
        wid = s * _NC + c

        @pl.when(wid < tiles)
        def _():
            off = wid * mp
            pltpu.sync_copy(mask.at[pl.ds(off, mp)], mb)
            a0 = pltpu.async_copy(q0.at[mb], r0, sem)
            a1 = pltpu.async_copy(q1.at[mb], r1, sem)
            a2 = pltpu.async_copy(lab.at[mb], lb, sem)
            a0.wait()
            a1.wait()

            def addb(e, carry):
                for j in range(d // 16):
                    r0[e, pl.ds(j * 16, 16)] = (r0[e, pl.ds(j * 16, 16)]
                                                + r1[e, pl.ds(j * 16, 16)])
                return carry
            lax.fori_loop(0, mp, addb, 0)

            pltpu.sync_copy(r0, md_out.at[pl.ds(off, mp)])
            a2.wait()
            pltpu.sync_copy(lb, ml_out.at[pl.ds(off, mp)])

    return mg


def kernel(x, edge_values, label, W1, W2, u_param, w_omega, b_omega, u_omega,
           edge_index, mask):
    n, d_in = x.shape
    d_out = W2.shape[1]
    n_edges = edge_index.shape[1]
    n_train = mask.shape[0]

    src = edge_index[0]
    dst2 = edge_index[1].reshape(-1, 80)
    labp = jnp.pad(label, ((0, 0), (0, 16 - label.shape[1])))

    xw1 = _matmul(x, W1, 1000)
    p1 = _make_segsum(n, n_edges, W1.shape[1])(xw1, src, dst2, edge_values)

    xw2 = pl.pallas_call(
        _norm_mm_body,
        out_shape=jax.ShapeDtypeStruct((n, d_out), jnp.float32),
    )(p1, W2)

    p2 = _make_segsum(n, n_edges, d_out)(xw2, src, dst2, edge_values)

    md, ml = _make_maskgather(n, n_train, d_out, 16)(p2[0], p2[1], labp, mask)

    loss, acc = pl.pallas_call(
        _tail_body,
        out_shape=(jax.ShapeDtypeStruct((1, 1), jnp.float32),
                   jax.ShapeDtypeStruct((1, 1), jnp.float32)),
        out_specs=(pl.BlockSpec(memory_space=pltpu.SMEM),
                   pl.BlockSpec(memory_space=pltpu.SMEM)),
    )(md, ml, u_param)

    return loss[0, 0], acc[0, 0]


# speedup vs baseline: 7.9361x; 1.0717x over previous
import functools

import jax
import jax.numpy as jnp
from jax import lax
from jax.experimental import pallas as pl
from jax.experimental.pallas import tpu as pltpu
from jax.experimental.pallas import tpu_sc as plsc

_NC = 2
_NS = 16
_NW = _NC * _NS


def _mm_body(x_ref, w_ref, o_ref):
    o_ref[...] = jnp.dot(x_ref[...], w_ref[...],
                         preferred_element_type=jnp.float32)


def _matmul(x, w, block_rows):
    n, k = x.shape
    m = w.shape[1]
    return pl.pallas_call(
        _mm_body,
        grid=(n // block_rows,),
        in_specs=[pl.BlockSpec((block_rows, k), lambda i: (i, 0)),
                  pl.BlockSpec((k, m), lambda i: (0, 0))],
        out_specs=pl.BlockSpec((block_rows, m), lambda i: (i, 0)),
        out_shape=jax.ShapeDtypeStruct((n, m), jnp.float32),
    )(x, w)


def _norm_mm_body(p_ref, w_ref, o_ref):
    s = p_ref[0] + p_ref[1]
    mean = jnp.mean(s, axis=0, keepdims=True)
    var = jnp.mean(jnp.square(s - mean), axis=0, keepdims=True)
    h = jnp.maximum((s - mean) / jnp.sqrt(var + 0.001), 0.0)
    o_ref[...] = jnp.dot(h, w_ref[...], preferred_element_type=jnp.float32)


def _tail_body(md_ref, ml_ref, u_ref, loss_ref, acc_ref):
    md = md_ref[...]
    ml = ml_ref[...][:, :2]
    z = jnp.dot(md, u_ref[...], preferred_element_type=jnp.float32)
    zm = jnp.max(z, axis=1, keepdims=True)
    ez = jnp.exp(z - zm)
    sm = ez / jnp.sum(ez, axis=1, keepdims=True)
    loss_ref[0, 0] = -jnp.sum(jnp.log(jax.nn.sigmoid(ml * sm)))
    pred = sm[:, 1] > sm[:, 0]
    tru = ml[:, 1] > ml[:, 0]
    acc_ref[0, 0] = jnp.mean((pred == tru).astype(jnp.float32))


@functools.lru_cache(maxsize=None)
def _make_segsum(n_nodes, n_edges, d):
    epw = n_edges // _NW
    c_sz = 80
    nch = epw // c_sz
    rpt = 640
    rlast = n_nodes - 15 * rpt
    mesh = plsc.VectorSubcoreMesh(core_axis_name="c", subcore_axis_name="s")

    @functools.partial(
        pl.kernel,
        out_type=jax.ShapeDtypeStruct((_NC, n_nodes, d), jnp.float32),
        mesh=mesh,
        compiler_params=pltpu.CompilerParams(use_tc_tiling_on_sc=False),
        scratch_types=[
            pltpu.VMEM((8, c_sz), jnp.int32),
            pltpu.VMEM((4, c_sz), jnp.int32),
            pltpu.VMEM((4, c_sz), jnp.float32),
            pltpu.VMEM((4, c_sz, d), jnp.float32),
            pltpu.VMEM_SHARED((n_nodes, d), jnp.float32),
            [pltpu.SemaphoreType.DMA] * 8,
            [pltpu.SemaphoreType.DMA] * 4,
            [pltpu.SemaphoreType.DMA] * 4,
        ],
    )
    def seg(xw, src, dst2, ev, out, srcr, dstr, evr, rows, acc,
            isems, gsems, ssems):
        c = lax.axis_index("c")
        s = lax.axis_index("s")
        wid = s * _NC + c
        base = wid * epw

        def start_idx(k, r):
            pltpu.async_copy(src.at[pl.ds(base + k * c_sz, c_sz)],
                             srcr.at[r], isems[r])

        def wait_idx(k, r):
            pltpu.make_async_copy(src.at[pl.ds(base + k * c_sz, c_sz)],
                                  srcr.at[r], isems[r]).wait()

        def gather_descs(k, b, r):
            return (
                pltpu.make_async_copy(xw.at[srcr.at[r]], rows.at[b],
                                      gsems[b]),
                pltpu.make_async_copy(ev.at[pl.ds(base + k * c_sz, c_sz)],
                                     evr.at[b], gsems[b]),
                pltpu.make_async_copy(dst2.at[wid * nch + k], dstr.at[b],
                                      gsems[b]),
            )

        def start_gather(k, b, r):
            for dsc in gather_descs(k, b, r):
                dsc.start()

        def wait_gather(k, b, r):
            for dsc in gather_descs(k, b, r):
                dsc.wait()

        def start_scatter(k, b):
            pltpu.async_copy(rows.at[b], acc.at[dstr.at[b]], ssems[b],
                             add=True)

        def wait_scatter(k, b):
            pltpu.make_async_copy(rows.at[b], acc.at[dstr.at[b]],
                                  ssems[b]).wait()

        for k0 in range(6):
            start_idx(k0, k0)

        def zfill(i, carry):
            def zlane(j, carry2):
                rows[3, i, pl.ds(j * 16, 16)] = jnp.zeros((16,), jnp.float32)
                return carry2
            return lax.fori_loop(0, d // 16, zlane, carry)
        lax.fori_loop(0, c_sz, zfill, 0)

        nz = jnp.where(s < 15, rpt // c_sz, rlast // c_sz)

        def zcopy(i, carry):
            pltpu.sync_copy(rows.at[3],
                            acc.at[pl.ds(s * rpt + i * c_sz, c_sz)])
            return carry
        lax.fori_loop(0, nz, zcopy, 0)

        for k0 in range(3):
            wait_idx(k0, k0)
            start_gather(k0, k0, k0)
        plsc.subcore_barrier()

        def do_step(m, k):
            b = m % 4
            wait_gather(k, b, m)

            @pl.when(k >= 1)
            def _():
                wait_scatter(k - 1, (b - 1) % 4)

            @pl.when(k + 3 < nch)
            def _():
                wait_idx(k + 3, (m + 3) % 8)
                start_gather(k + 3, (b - 1) % 4, (m + 3) % 8)

            @pl.when(k + 6 < nch)
            def _():
                start_idx(k + 6, (m + 6) % 8)

            def escale(g, carry2):
                evs = evr[b, pl.ds(g * 16, 16)]
                for lane in range(16):
                    e = g * 16 + lane
                    bv = jnp.full((16,), evs[lane])
                    for j in range(d // 16):
                        rows[b, e, pl.ds(j * 16, 16)] = (
                            rows[b, e, pl.ds(j * 16, 16)] * bv)
                return carry2
            lax.fori_loop(0, c_sz // 16, escale, 0)

            start_scatter(k, b)

        def step(k, carry):
            for m in range(8):
                @pl.when(k % 8 == m)
                def _(m=m):
                    do_step(m, k)
            return carry
        lax.fori_loop(0, nch, step, 0)

        wait_scatter(nch - 1, (nch - 1) % 4)
        plsc.subcore_barrier()

        @pl.when(s < 15)
        def _():
            pltpu.sync_copy(acc.at[pl.ds(s * rpt, rpt)],
                            out.at[c, pl.ds(s * rpt, rpt)])

        @pl.when(s == 15)
        def _():
            pltpu.sync_copy(acc.at[pl.ds(15 * rpt, rlast)],
                            out.at[c, pl.ds(15 * rpt, rlast)])

    return seg


@functools.lru_cache(maxsize=None)
def _make_maskgather(n_nodes, n_train, d, dl):
    tiles = 25
    mp = n_train // tiles
    mesh = plsc.VectorSubcoreMesh(core_axis_name="c", subcore_axis_name="s")

    @functools.partial(
        pl.kernel,
        out_type=(jax.ShapeDtypeStruct((n_train, d), jnp.float32),
                  jax.ShapeDtypeStruct((n_train, dl), jnp.float32)),
        mesh=mesh,
        compiler_params=pltpu.CompilerParams(use_tc_tiling_on_sc=False),
        scratch_types=[
            pltpu.VMEM((mp,), jnp.int32),
            pltpu.VMEM((mp, d), jnp.float32),
            pltpu.VMEM((mp, d), jnp.float32),
            pltpu.VMEM((mp, dl), jnp.float32),
            pltpu.SemaphoreType.DMA,
        ],
    )
    def mg(q0, q1, lab, mask, md_out, ml_out, mb, r0, r1, lb, sem):
        c = lax.axis_index("c")
        s = lax.axis_index("s")
        wid = s * _NC + c

        @pl.when(wid < tiles)
        def _():
            off = wid * mp
            pltpu.sync_copy(mask.at[pl.ds(off, mp)], mb)
            a0 = pltpu.async_copy(q0.at[mb], r0, sem)
            a1 = pltpu.async_copy(q1.at[mb], r1, sem)
            a2 = pltpu.async_copy(lab.at[mb], lb, sem)
            a0.wait()
            a1.wait()

            def addb(e, carry):
                for j in range(d // 16):
                    r0[e, pl.ds(j * 16, 16)] = (r0[e, pl.ds(j * 16, 16)]
                                                + r1[e, pl.ds(j * 16, 16)])
                return carry
            lax.fori_loop(0, mp, addb, 0)

            pltpu.sync_copy(r0, md_out.at[pl.ds(off, mp)])
            a2.wait()
            pltpu.sync_copy(lb, ml_out.at[pl.ds(off, mp)])

    return mg


def kernel(x, edge_values, label, W1, W2, u_param, w_omega, b_omega, u_omega,
           edge_index, mask):
    n, d_in = x.shape
    d_out = W2.shape[1]
    n_edges = edge_index.shape[1]
    n_train = mask.shape[0]

    src = edge_index[0]
    dst2 = edge_index[1].reshape(-1, 80)
    labp = jnp.pad(label, ((0, 0), (0, 16 - label.shape[1])))

    xw1 = _matmul(x, W1, 1000)
    p1 = _make_segsum(n, n_edges, W1.shape[1])(xw1, src, dst2, edge_values)

    xw2 = pl.pallas_call(
        _norm_mm_body,
        out_shape=jax.ShapeDtypeStruct((n, d_out), jnp.float32),
    )(p1, W2)

    p2 = _make_segsum(n, n_edges, d_out)(xw2, src, dst2, edge_values)

    md, ml = _make_maskgather(n, n_train, d_out, 16)(p2[0], p2[1], labp, mask)

    loss, acc = pl.pallas_call(
        _tail_body,
        out_shape=(jax.ShapeDtypeStruct((1, 1), jnp.float32),
                   jax.ShapeDtypeStruct((1, 1), jnp.float32)),
        out_specs=(pl.BlockSpec(memory_space=pltpu.SMEM),
                   pl.BlockSpec(memory_space=pltpu.SMEM)),
    )(md, ml, u_param)

    return loss[0, 0], acc[0, 0]


# speedup vs baseline: 12.2337x; 1.5415x over previous
import functools

import jax
import jax.numpy as jnp
from jax import lax
from jax.experimental import pallas as pl
from jax.experimental.pallas import tpu as pltpu
from jax.experimental.pallas import tpu_sc as plsc

_NC = 2
_NS = 16
_NW = _NC * _NS


def _mm_body(x_ref, w_ref, o_ref):
    o_ref[...] = jnp.dot(x_ref[...], w_ref[...],
                         preferred_element_type=jnp.float32)


def _matmul(x, w, block_rows):
    n, k = x.shape
    m = w.shape[1]
    return pl.pallas_call(
        _mm_body,
        grid=(n // block_rows,),
        in_specs=[pl.BlockSpec((block_rows, k), lambda i: (i, 0)),
                  pl.BlockSpec((k, m), lambda i: (0, 0))],
        out_specs=pl.BlockSpec((block_rows, m), lambda i: (i, 0)),
        out_shape=jax.ShapeDtypeStruct((n, m), jnp.float32),
    )(x, w)


def _norm_mm_body(p_ref, w_ref, o_ref):
    s = p_ref[0] + p_ref[1]
    mean = jnp.mean(s, axis=0, keepdims=True)
    var = jnp.mean(jnp.square(s - mean), axis=0, keepdims=True)
    h = jnp.maximum((s - mean) / jnp.sqrt(var + 0.001), 0.0)
    o_ref[...] = jnp.dot(h, w_ref[...], preferred_element_type=jnp.float32)


def _tail_body(md_ref, ml_ref, u_ref, loss_ref, acc_ref):
    md = md_ref[...]
    ml = ml_ref[...][:, :2]
    z = jnp.dot(md, u_ref[...], preferred_element_type=jnp.float32)
    zm = jnp.max(z, axis=1, keepdims=True)
    ez = jnp.exp(z - zm)
    sm = ez / jnp.sum(ez, axis=1, keepdims=True)
    loss_ref[0, 0] = -jnp.sum(jnp.log(jax.nn.sigmoid(ml * sm)))
    pred = sm[:, 1] > sm[:, 0]
    tru = ml[:, 1] > ml[:, 0]
    acc_ref[0, 0] = jnp.mean((pred == tru).astype(jnp.float32))


@functools.lru_cache(maxsize=None)
def _make_segsum(n_nodes, n_edges, d):
    epw = n_edges // _NW
    c_sz = 80
    nch = epw // c_sz
    rpt = 640
    rlast = n_nodes - 15 * rpt
    mesh = plsc.VectorSubcoreMesh(core_axis_name="c", subcore_axis_name="s")

    @functools.partial(
        pl.kernel,
        out_type=jax.ShapeDtypeStruct((_NC, n_nodes, d), jnp.float32),
        mesh=mesh,
        compiler_params=pltpu.CompilerParams(use_tc_tiling_on_sc=False),
        scratch_types=[
            pltpu.VMEM((8, c_sz), jnp.int32),
            pltpu.VMEM((4, c_sz), jnp.int32),
            pltpu.VMEM((4, c_sz), jnp.float32),
            pltpu.VMEM((4, c_sz, d), jnp.float32),
            pltpu.VMEM_SHARED((n_nodes, d), jnp.float32),
            [pltpu.SemaphoreType.DMA] * 8,
            [pltpu.SemaphoreType.DMA] * 4,
            [pltpu.SemaphoreType.DMA] * 4,
        ],
    )
    def seg(xw, src, dst2, ev, out, srcr, dstr, evr, rows, acc,
            isems, gsems, ssems):
        c = lax.axis_index("c")
        s = lax.axis_index("s")
        wid = s * _NC + c
        base = wid * epw

        def start_idx(k, r):
            pltpu.async_copy(src.at[pl.ds(base + k * c_sz, c_sz)],
                             srcr.at[r], isems[r])

        def wait_idx(k, r):
            pltpu.make_async_copy(src.at[pl.ds(base + k * c_sz, c_sz)],
                                  srcr.at[r], isems[r]).wait()

        def gather_descs(k, b, r):
            return (
                pltpu.make_async_copy(xw.at[srcr.at[r]], rows.at[b],
                                      gsems[b]),
                pltpu.make_async_copy(ev.at[pl.ds(base + k * c_sz, c_sz)],
                                     evr.at[b], gsems[b]),
                pltpu.make_async_copy(dst2.at[wid * nch + k], dstr.at[b],
                                      gsems[b]),
            )

        def start_gather(k, b, r):
            for dsc in gather_descs(k, b, r):
                dsc.start()

        def wait_gather(k, b, r):
            for dsc in gather_descs(k, b, r):
                dsc.wait()

        def start_scatter(k, b):
            pltpu.async_copy(rows.at[b], acc.at[dstr.at[b]], ssems[b],
                             add=True)

        def wait_scatter(k, b):
            pltpu.make_async_copy(rows.at[b], acc.at[dstr.at[b]],
                                  ssems[b]).wait()

        for k0 in range(6):
            start_idx(k0, k0)

        def zfill(i, carry):
            def zlane(j, carry2):
                rows[3, i, pl.ds(j * 16, 16)] = jnp.zeros((16,), jnp.float32)
                return carry2
            return lax.fori_loop(0, d // 16, zlane, carry)
        lax.fori_loop(0, c_sz, zfill, 0)

        nz = jnp.where(s < 15, rpt // c_sz, rlast // c_sz)

        def zcopy(i, carry):
            pltpu.sync_copy(rows.at[3],
                            acc.at[pl.ds(s * rpt + i * c_sz, c_sz)])
            return carry
        lax.fori_loop(0, nz, zcopy, 0)

        for k0 in range(3):
            wait_idx(k0, k0)
            start_gather(k0, k0, k0)
        plsc.subcore_barrier()

        def do_step(m, k):
            b = m % 4
            wait_gather(k, b, m)

            @pl.when(k >= 1)
            def _():
                wait_scatter(k - 1, (b - 1) % 4)

            @pl.when(k + 3 < nch)
            def _():
                wait_idx(k + 3, (m + 3) % 8)
                start_gather(k + 3, (b - 1) % 4, (m + 3) % 8)

            @pl.when(k + 6 < nch)
            def _():
                start_idx(k + 6, (m + 6) % 8)

            def escale(g, carry2):
                evs = evr[b, pl.ds(g * 16, 16)]
                for lane in range(16):
                    e = g * 16 + lane
                    bv = jnp.full((16,), evs[lane])
                    for j in range(d // 16):
                        rows[b, e, pl.ds(j * 16, 16)] = (
                            rows[b, e, pl.ds(j * 16, 16)] * bv)
                return carry2
            lax.fori_loop(0, c_sz // 16, escale, 0)

            start_scatter(k, b)

        def step(k, carry):
            for m in range(8):
                @pl.when(k % 8 == m)
                def _(m=m):
                    do_step(m, k)
            return carry
        lax.fori_loop(0, nch, step, 0)

        wait_scatter(nch - 1, (nch - 1) % 4)
        plsc.subcore_barrier()

        @pl.when(s < 15)
        def _():
            pltpu.sync_copy(acc.at[pl.ds(s * rpt, rpt)],
                            out.at[c, pl.ds(s * rpt, rpt)])

        @pl.when(s == 15)
        def _():
            pltpu.sync_copy(acc.at[pl.ds(15 * rpt, rlast)],
                            out.at[c, pl.ds(15 * rpt, rlast)])

    return seg


@functools.lru_cache(maxsize=None)
def _make_compact(n_nodes, n_edges, n_train):
    epw = n_edges // _NW
    c_sz = 80
    grp = epw // 16
    mgrp = n_train // 16
    tiles = 25
    mp = n_train // tiles
    mesh = plsc.VectorSubcoreMesh(core_axis_name="c", subcore_axis_name="s")

    @functools.partial(
        pl.kernel,
        out_type=(jax.ShapeDtypeStruct((_NW, epw), jnp.int32),
                  jax.ShapeDtypeStruct((_NW, epw), jnp.float32),
                  jax.ShapeDtypeStruct((_NW, epw), jnp.int32),
                  jax.ShapeDtypeStruct((_NW, 16), jnp.int32),
                  jax.ShapeDtypeStruct((n_train,), jnp.int32)),
        mesh=mesh,
        compiler_params=pltpu.CompilerParams(use_tc_tiling_on_sc=False,
                                             needs_layout_passes=False),
        scratch_types=[
            pltpu.VMEM((n_train,), jnp.int32),
            pltpu.VMEM((n_nodes,), jnp.int32),
            pltpu.VMEM((epw,), jnp.int32),
            pltpu.VMEM((epw,), jnp.int32),
            pltpu.VMEM((epw,), jnp.float32),
            pltpu.VMEM((epw,), jnp.int32),
            pltpu.VMEM((epw,), jnp.float32),
            pltpu.VMEM((epw,), jnp.int32),
            pltpu.VMEM((16,), jnp.int32),
            pltpu.VMEM((mp,), jnp.int32),
            pltpu.SemaphoreType.DMA,
        ],
    )
    def comp(src, dst, ev, mask, csrc_o, cev_o, cslot_o, cnt_o, slot_o,
             maskv, pos, srcv, dstv, evv, csv, cev, csl, cntv, slv, lsem):
        c = lax.axis_index("c")
        s = lax.axis_index("s")
        wid = s * _NC + c
        base = wid * epw

        l0 = pltpu.async_copy(src.at[pl.ds(base, epw)], srcv, lsem)
        l1 = pltpu.async_copy(dst.at[pl.ds(base, epw)], dstv, lsem)
        l2 = pltpu.async_copy(ev.at[pl.ds(base, epw)], evv, lsem)
        pltpu.sync_copy(mask, maskv)

        zero_i = jnp.zeros((16,), jnp.int32)
        zero_f = jnp.zeros((16,), jnp.float32)
        neg1 = jnp.full((16,), -1, jnp.int32)

        def zinit(i, carry):
            pos[pl.ds(i * 16, 16)] = neg1
            return carry
        lax.fori_loop(0, n_nodes // 16, zinit, 0)

        def zbuf(i, carry):
            csv[pl.ds(i * 16, 16)] = zero_i
            cev[pl.ds(i * 16, 16)] = zero_f
            csl[pl.ds(i * 16, 16)] = zero_i
            return carry
        lax.fori_loop(0, grp, zbuf, 0)

        lanes = lax.iota(jnp.int32, 16)

        def build(g, carry):
            idx = maskv[pl.ds(g * 16, 16)]
            plsc.store_scatter(pos, [idx], lanes + g * 16)
            return carry
        lax.fori_loop(0, mgrp, build, 0)

        l0.wait()
        l1.wait()
        l2.wait()

        def ckeep(g, cnt):
            d16 = dstv[pl.ds(g * 16, 16)]
            p16 = plsc.load_gather(pos, [d16])
            m = p16 >= 0
            plsc.store_compressed(csv.at[pl.ds(cnt, 16)],
                                  srcv[pl.ds(g * 16, 16)], mask=m)
            plsc.store_compressed(cev.at[pl.ds(cnt, 16)],
                                  evv[pl.ds(g * 16, 16)], mask=m)
            plsc.store_compressed(csl.at[pl.ds(cnt, 16)], p16, mask=m)
            return cnt + plsc.all_reduce_population_count(m)[0]
        cnt = lax.fori_loop(0, grp, ckeep, jnp.int32(0))

        nch_c = jnp.maximum((cnt + (c_sz - 1)) // c_sz, 3)
        cntv[...] = jnp.full((16,), nch_c, jnp.int32)

        pltpu.sync_copy(csv, csrc_o.at[wid])
        pltpu.sync_copy(cev, cev_o.at[wid])
        pltpu.sync_copy(csl, cslot_o.at[wid])
        pltpu.sync_copy(cntv, cnt_o.at[wid])

        @pl.when(wid < tiles)
        def _():
            def sgrp(g, carry):
                mk = maskv[pl.ds(wid * mp + g * 16, 16)]
                slv[pl.ds(g * 16, 16)] = plsc.load_gather(pos, [mk])
                return carry
            lax.fori_loop(0, mp // 16, sgrp, 0)
            pltpu.sync_copy(slv, slot_o.at[pl.ds(wid * mp, mp)])

    return comp


@functools.lru_cache(maxsize=None)
def _make_segsum_small(n_nodes, n_edges, n_slots, d):
    epw = n_edges // _NW
    c_sz = 80
    nch_max = epw // c_sz
    rpt = n_slots // _NS
    mesh = plsc.VectorSubcoreMesh(core_axis_name="c", subcore_axis_name="s")

    @functools.partial(
        pl.kernel,
        out_type=jax.ShapeDtypeStruct((_NC, n_slots, d), jnp.float32),
        mesh=mesh,
        compiler_params=pltpu.CompilerParams(use_tc_tiling_on_sc=False),
        scratch_types=[
            pltpu.VMEM((epw,), jnp.int32),
            pltpu.VMEM((epw,), jnp.float32),
            pltpu.VMEM((4, c_sz), jnp.int32),
            pltpu.VMEM((4, c_sz, d), jnp.float32),
            pltpu.VMEM_SHARED((n_slots, d), jnp.float32),
            pltpu.SemaphoreType.DMA,
            [pltpu.SemaphoreType.DMA] * 4,
            [pltpu.SemaphoreType.DMA] * 4,
        ],
    )
    def seg(xw, csrc, cev, cslot3, cnt_in, out, srcv, evv, dstr, rows, acc,
            lsem, gsems, ssems):
        c = lax.axis_index("c")
        s = lax.axis_index("s")
        wid = s * _NC + c

        l0 = pltpu.async_copy(csrc.at[wid], srcv, lsem)
        l1 = pltpu.async_copy(cev.at[wid], evv, lsem)

        pltpu.sync_copy(cnt_in.at[wid], dstr.at[0, pl.ds(0, 16)])
        nch = dstr[0, pl.ds(0, 16)][0]

        def gather_descs(k, b):
            return (
                pltpu.make_async_copy(xw.at[srcv.at[pl.ds(k * c_sz, c_sz)]],
                                      rows.at[b], gsems[b]),
                pltpu.make_async_copy(cslot3.at[wid, k], dstr.at[b],
                                      gsems[b]),
            )

        def start_gather(k, b):
            for dsc in gather_descs(k, b):
                dsc.start()

        def wait_gather(k, b):
            for dsc in gather_descs(k, b):
                dsc.wait()

        def start_scatter(k, b):
            pltpu.async_copy(rows.at[b], acc.at[dstr.at[b]], ssems[b],
                             add=True)

        def wait_scatter(k, b):
            pltpu.make_async_copy(rows.at[b], acc.at[dstr.at[b]],
                                  ssems[b]).wait()

        def zfill(i, carry):
            def zlane(j, carry2):
                rows[3, i, pl.ds(j * 16, 16)] = jnp.zeros((16,), jnp.float32)
                return carry2
            return lax.fori_loop(0, d // 16, zlane, carry)
        lax.fori_loop(0, c_sz, zfill, 0)
        pltpu.sync_copy(rows.at[3], acc.at[pl.ds(s * rpt, c_sz)])
        pltpu.sync_copy(rows.at[3, pl.ds(0, rpt - c_sz)],
                        acc.at[pl.ds(s * rpt + c_sz, rpt - c_sz)])

        l0.wait()
        l1.wait()
        for k0 in range(3):
            start_gather(k0, k0)
        plsc.subcore_barrier()

        def do_step(b, k):
            wait_gather(k, b)

            @pl.when(k >= 1)
            def _():
                wait_scatter(k - 1, (b - 1) % 4)

            @pl.when(k + 3 < nch)
            def _():
                start_gather(k + 3, (b - 1) % 4)

            def escale(g, carry2):
                evs = evv[pl.ds(k * c_sz + g * 16, 16)]
                for lane in range(16):
                    e = g * 16 + lane
                    bv = jnp.full((16,), evs[lane])
                    for j in range(d // 16):
                        rows[b, e, pl.ds(j * 16, 16)] = (
                            rows[b, e, pl.ds(j * 16, 16)] * bv)
                return carry2
            lax.fori_loop(0, c_sz // 16, escale, 0)

            start_scatter(k, b)

        def step(k, carry):
            for b in range(4):
                @pl.when(k % 4 == b)
                def _(b=b):
                    do_step(b, k)
            return carry
        lax.fori_loop(0, nch, step, 0)

        last = nch - 1
        for b in range(4):
            @pl.when(last % 4 == b)
            def _(b=b):
                wait_scatter(last, b)
        plsc.subcore_barrier()

        pltpu.sync_copy(acc.at[pl.ds(s * rpt, rpt)],
                        out.at[c, pl.ds(s * rpt, rpt)])

    return seg


@functools.lru_cache(maxsize=None)
def _make_maskgather(n_nodes, n_slots, n_train, d, dl):
    tiles = 25
    mp = n_train // tiles
    mesh = plsc.VectorSubcoreMesh(core_axis_name="c", subcore_axis_name="s")

    @functools.partial(
        pl.kernel,
        out_type=(jax.ShapeDtypeStruct((n_train, d), jnp.float32),
                  jax.ShapeDtypeStruct((n_train, dl), jnp.float32)),
        mesh=mesh,
        compiler_params=pltpu.CompilerParams(use_tc_tiling_on_sc=False),
        scratch_types=[
            pltpu.VMEM((mp,), jnp.int32),
            pltpu.VMEM((mp,), jnp.int32),
            pltpu.VMEM((mp, d), jnp.float32),
            pltpu.VMEM((mp, d), jnp.float32),
            pltpu.VMEM((mp, dl), jnp.float32),
            pltpu.SemaphoreType.DMA,
        ],
    )
    def mg(q0, q1, lab, mask, slot, md_out, ml_out, mb, sb, r0, r1, lb, sem):
        c = lax.axis_index("c")
        s = lax.axis_index("s")
        wid = s * _NC + c

        @pl.when(wid < tiles)
        def _():
            off = wid * mp
            pltpu.sync_copy(mask.at[pl.ds(off, mp)], mb)
            pltpu.sync_copy(slot.at[pl.ds(off, mp)], sb)
            a0 = pltpu.async_copy(q0.at[sb], r0, sem)
            a1 = pltpu.async_copy(q1.at[sb], r1, sem)
            a2 = pltpu.async_copy(lab.at[mb], lb, sem)
            a0.wait()
            a1.wait()

            def addb(e, carry):
                for j in range(d // 16):
                    r0[e, pl.ds(j * 16, 16)] = (r0[e, pl.ds(j * 16, 16)]
                                                + r1[e, pl.ds(j * 16, 16)])
                return carry
            lax.fori_loop(0, mp, addb, 0)

            pltpu.sync_copy(r0, md_out.at[pl.ds(off, mp)])
            a2.wait()
            pltpu.sync_copy(lb, ml_out.at[pl.ds(off, mp)])

    return mg


def kernel(x, edge_values, label, W1, W2, u_param, w_omega, b_omega, u_omega,
           edge_index, mask):
    n, d_in = x.shape
    d_out = W2.shape[1]
    n_edges = edge_index.shape[1]
    n_train = mask.shape[0]

    src = edge_index[0]
    dst2 = edge_index[1].reshape(-1, 80)
    labp = jnp.pad(label, ((0, 0), (0, 16 - label.shape[1])))

    xw1 = _matmul(x, W1, 1000)
    p1 = _make_segsum(n, n_edges, W1.shape[1])(xw1, src, dst2, edge_values)

    xw2 = pl.pallas_call(
        _norm_mm_body,
        out_shape=jax.ShapeDtypeStruct((n, d_out), jnp.float32),
    )(p1, W2)

    csrc, cev, cslot, ccnt, slotvec = _make_compact(n, n_edges, n_train)(
        src, edge_index[1], edge_values, mask)
    cslot3 = cslot.reshape(_NW, -1, 80)
    n_slots = 2048
    p2 = _make_segsum_small(n, n_edges, n_slots, d_out)(
        xw2, csrc, cev, cslot3, ccnt)

    md, ml = _make_maskgather(n, n_slots, n_train, d_out, 16)(
        p2[0], p2[1], labp, mask, slotvec)

    loss, acc = pl.pallas_call(
        _tail_body,
        out_shape=(jax.ShapeDtypeStruct((1, 1), jnp.float32),
                   jax.ShapeDtypeStruct((1, 1), jnp.float32)),
        out_specs=(pl.BlockSpec(memory_space=pltpu.SMEM),
                   pl.BlockSpec(memory_space=pltpu.SMEM)),
    )(md, ml, u_param)

    return loss[0, 0], acc[0, 0]
